# baseline (speedup 1.0000x reference)
import functools

import jax
import jax.numpy as jnp
from jax import lax
from jax.experimental import pallas as pl
from jax.experimental.pallas import tpu as pltpu
from jax.experimental.pallas import tpu_sc as plsc

_NC = 2
_NS = 16
_NW = _NC * _NS
_L = 16

_CHUNK_ROWS = 8


def _make_sc_add(batch, seq, d):
    rows_per_w = seq // _NW
    n_chunks = rows_per_w // _CHUNK_ROWS
    cw = _CHUNK_ROWS * d

    mesh = plsc.VectorSubcoreMesh(core_axis_name="c", subcore_axis_name="s")

    @functools.partial(
        pl.kernel,
        mesh=mesh,
        out_type=jax.ShapeDtypeStruct((batch, seq * d), jnp.float32),
        scratch_types=[pltpu.VMEM((cw,), jnp.float32)]
        + [pltpu.VMEM((cw,), jnp.float32) for _ in range(batch)],
    )
    def sc_add(x_hbm, pos_hbm, out_hbm, pos_v, *x_v):
        wid = lax.axis_index("s") * _NC + lax.axis_index("c")
        base = wid * (rows_per_w * d)
        for c in range(n_chunks):
            off = base + c * cw
            pltpu.sync_copy(pos_hbm.at[pl.ds(off, cw)], pos_v)
            for b in range(batch):
                pltpu.sync_copy(x_hbm.at[b, pl.ds(off, cw)], x_v[b])

            def body(i, carry):
                sl = pl.ds(i * _L, _L)
                p = pos_v[sl]
                for b in range(batch):
                    x_v[b][sl] = x_v[b][sl] + p
                return carry

            lax.fori_loop(0, cw // _L, body, 0)
            for b in range(batch):
                pltpu.sync_copy(x_v[b], out_hbm.at[b, pl.ds(off, cw)])

    return sc_add


def kernel(x, pos_table):
    batch, seq, d = x.shape
    pos = pos_table[:seq]
    x2 = x.reshape(batch, seq * d)
    pos1 = pos.reshape(seq * d)
    out = _make_sc_add(batch, seq, d)(x2, pos1)
    return out.reshape(batch, seq, d)


# speedup vs baseline: 1.3153x; 1.3153x over previous
import functools

import jax
import jax.numpy as jnp
from jax import lax
from jax.experimental import pallas as pl
from jax.experimental.pallas import tpu as pltpu
from jax.experimental.pallas import tpu_sc as plsc

_NC = 2
_NS = 16
_NW = _NC * _NS
_L = 16

_CHUNK_ROWS = 8


def _make_sc_add(batch, seq, d):
    rows_per_w = seq // _NW
    n_chunks = rows_per_w // _CHUNK_ROWS
    cw = _CHUNK_ROWS * d

    mesh = plsc.VectorSubcoreMesh(core_axis_name="c", subcore_axis_name="s")

    vmem = [pltpu.VMEM((cw,), jnp.float32) for _ in range(2 * (batch + 1))]
    sems = [pltpu.SemaphoreType.DMA for _ in range(4)]

    @functools.partial(
        pl.kernel,
        mesh=mesh,
        out_type=jax.ShapeDtypeStruct((batch, seq * d), jnp.float32),
        scratch_types=vmem + sems,
    )
    def sc_add(x_hbm, pos_hbm, out_hbm, *scratch):
        bufs, sem4 = scratch[: 2 * (batch + 1)], scratch[2 * (batch + 1) :]
        pos_v = (bufs[0], bufs[batch + 1])
        x_v = (bufs[1 : batch + 1], bufs[batch + 2 : 2 * (batch + 1)])
        in_sem = (sem4[0], sem4[1])
        out_sem = (sem4[2], sem4[3])

        wid = lax.axis_index("s") * _NC + lax.axis_index("c")
        base = wid * (rows_per_w * d)

        def start_in(c, slot):
            off = base + c * cw
            hs = [pltpu.async_copy(pos_hbm.at[pl.ds(off, cw)], pos_v[slot],
                                   in_sem[slot])]
            for b in range(batch):
                hs.append(pltpu.async_copy(x_hbm.at[b, pl.ds(off, cw)],
                                           x_v[slot][b], in_sem[slot]))
            return hs

        in_handles = [None, None]
        out_handles = [None, None]
        in_handles[0] = start_in(0, 0)
        for c in range(n_chunks):
            slot = c % 2
            nxt = 1 - slot
            if c + 1 < n_chunks:
                if out_handles[nxt] is not None:
                    for h in out_handles[nxt]:
                        h.wait()
                    out_handles[nxt] = None
                in_handles[nxt] = start_in(c + 1, nxt)
            for h in in_handles[slot]:
                h.wait()

            def body(i, carry):
                sl = pl.ds(i * _L, _L)
                p = pos_v[slot][sl]
                for b in range(batch):
                    x_v[slot][b][sl] = x_v[slot][b][sl] + p
                return carry

            lax.fori_loop(0, cw // _L, body, 0)

            off = base + c * cw
            out_handles[slot] = [
                pltpu.async_copy(x_v[slot][b], out_hbm.at[b, pl.ds(off, cw)],
                                 out_sem[slot])
                for b in range(batch)
            ]
        for hs in out_handles:
            if hs is not None:
                for h in hs:
                    h.wait()

    return sc_add


def kernel(x, pos_table):
    batch, seq, d = x.shape
    pos = pos_table[:seq]
    x2 = x.reshape(batch, seq * d)
    pos1 = pos.reshape(seq * d)
    out = _make_sc_add(batch, seq, d)(x2, pos1)
    return out.reshape(batch, seq, d)


# speedup vs baseline: 2.9697x; 2.2578x over previous
import functools

import jax
import jax.numpy as jnp
from jax import lax
from jax.experimental import pallas as pl
from jax.experimental.pallas import tpu as pltpu
from jax.experimental.pallas import tpu_sc as plsc

_NC = 2
_NS = 16
_NW = _NC * _NS
_L = 16

_CHUNK_ROWS = 8


def _make_sc_add(batch, seq, d):
    rows_per_w = seq // _NW
    n_chunks = rows_per_w // _CHUNK_ROWS
    n_col = d // _L

    mesh = plsc.VectorSubcoreMesh(core_axis_name="c", subcore_axis_name="s")

    vmem = [pltpu.VMEM((_CHUNK_ROWS, d), jnp.float32)
            for _ in range(2 * (batch + 1))]
    sems = [pltpu.SemaphoreType.DMA for _ in range(4)]

    @functools.partial(
        pl.kernel,
        mesh=mesh,
        out_type=jax.ShapeDtypeStruct((batch, seq, d), jnp.float32),
        scratch_types=vmem + sems,
    )
    def sc_add(x_hbm, pos_hbm, out_hbm, *scratch):
        bufs, sem4 = scratch[: 2 * (batch + 1)], scratch[2 * (batch + 1) :]
        pos_v = (bufs[0], bufs[batch + 1])
        x_v = (bufs[1 : batch + 1], bufs[batch + 2 : 2 * (batch + 1)])
        in_sem = (sem4[0], sem4[1])
        out_sem = (sem4[2], sem4[3])

        wid = lax.axis_index("s") * _NC + lax.axis_index("c")
        base = wid * rows_per_w

        def start_in(c, slot):
            row = base + c * _CHUNK_ROWS
            hs = [pltpu.async_copy(pos_hbm.at[pl.ds(row, _CHUNK_ROWS), :],
                                   pos_v[slot], in_sem[slot])]
            for b in range(batch):
                hs.append(pltpu.async_copy(
                    x_hbm.at[b, pl.ds(row, _CHUNK_ROWS), :],
                    x_v[slot][b], in_sem[slot]))
            return hs

        in_handles = [None, None]
        out_handles = [None, None]
        in_handles[0] = start_in(0, 0)
        for c in range(n_chunks):
            slot = c % 2
            nxt = 1 - slot
            if c + 1 < n_chunks:
                if out_handles[nxt] is not None:
                    for h in out_handles[nxt]:
                        h.wait()
                    out_handles[nxt] = None
                in_handles[nxt] = start_in(c + 1, nxt)
            for h in in_handles[slot]:
                h.wait()

            def body(i, carry):
                r = i >> 6 if n_col == 64 else i // n_col
                col = (i - r * n_col) * _L
                sl = pl.ds(col, _L)
                p = pos_v[slot][r, sl]
                for b in range(batch):
                    x_v[slot][b][r, sl] = x_v[slot][b][r, sl] + p
                return carry

            lax.fori_loop(0, _CHUNK_ROWS * n_col, body, 0)

            row = base + c * _CHUNK_ROWS
            out_handles[slot] = [
                pltpu.async_copy(x_v[slot][b],
                                 out_hbm.at[b, pl.ds(row, _CHUNK_ROWS), :],
                                 out_sem[slot])
                for b in range(batch)
            ]
        for hs in out_handles:
            if hs is not None:
                for h in hs:
                    h.wait()

    return sc_add


def kernel(x, pos_table):
    batch, seq, d = x.shape
    pos = pos_table[:seq]
    return _make_sc_add(batch, seq, d)(x, pos)
